# Initial kernel scaffold; baseline (speedup 1.0000x reference)
import jax
import jax.numpy as jnp
from jax.experimental import pallas as pl


def kernel(W1, W2, src, rel, dst):
    raise NotImplementedError("write your pallas kernel here")


# speedup vs baseline: 70.6561x; 70.6561x over previous
import functools

import jax
import jax.numpy as jnp
from jax import lax
from jax.experimental import pallas as pl
from jax.experimental.pallas import tpu as pltpu
from jax.experimental.pallas import tpu_sc as plsc

F = 16
NS = 16
NC = 2
SUB = 80
NSUB = 25
CH = SUB * NSUB
GRP = CH // 16


def _norm_body(E, N, NRL, DEGP,
               src_h, rel_h, dst_h, norm_h,
               deg_sh, sbuf, rbuf, dbuf, kbuf, nbuf, ones_v, work_v, sem):
    c = lax.axis_index("c")
    s = lax.axis_index("s")
    iot = lax.iota(jnp.int32, 16)
    slc = DEGP // NS
    epw = E // NS
    eps = (2 * E) // NS

    def fz(g, _):
        work_v[pl.ds(g * 16, 16)] = jnp.zeros((16,), jnp.float32)
        return _
    lax.fori_loop(0, slc // 16, fz, None)
    for j in range(SUB // 16):
        ones_v[pl.ds(j * 16, 16)] = jnp.ones((16,), jnp.float32)
    pltpu.sync_copy(work_v, deg_sh.at[pl.ds(s * slc, slc)])
    plsc.subcore_barrier()

    fwd = s < 8
    base0 = jnp.where(fwd, s, s - 8) * eps

    def deg_chunk(k, _):
        base = base0 + k * CH
        pltpu.sync_copy(src_h.at[pl.ds(base, CH)], sbuf)
        pltpu.sync_copy(rel_h.at[pl.ds(base, CH)], rbuf)
        pltpu.sync_copy(dst_h.at[pl.ds(base, CH)], dbuf)

        def cg(g, _2):
            j = g // (SUB // 16)
            col = (g % (SUB // 16)) * 16
            sv = sbuf[pl.ds(g * 16, 16)]
            rv = rbuf[pl.ds(g * 16, 16)]
            dv = dbuf[pl.ds(g * 16, 16)]
            ov = jnp.where(fwd, dv, sv)
            r8 = rv + jnp.where(fwd, 0, NRL)
            kbuf[j, pl.ds(col, 16)] = r8 * N + ov
            return _2
        lax.fori_loop(0, GRP, cg, None)
        for j in range(NSUB):
            pltpu.sync_copy(ones_v, deg_sh.at[kbuf.at[j]], add=True)
        return _
    lax.fori_loop(0, eps // CH, deg_chunk, None)
    plsc.subcore_barrier()

    off = s * slc
    pltpu.sync_copy(deg_sh.at[pl.ds(off, slc)], work_v)

    def inv_g(g, _):
        v = work_v[pl.ds(g * 16, 16)]
        work_v[pl.ds(g * 16, 16)] = 1.0 / jnp.maximum(v, 1.0)
        return _
    lax.fori_loop(0, slc // 16, inv_g, None)
    pltpu.sync_copy(work_v, deg_sh.at[pl.ds(off, slc)])
    plsc.subcore_barrier()

    def norm_chunk(k, _):
        base = s * epw + k * CH
        pltpu.sync_copy(src_h.at[pl.ds(base, CH)], sbuf)
        pltpu.sync_copy(rel_h.at[pl.ds(base, CH)], rbuf)
        pltpu.sync_copy(dst_h.at[pl.ds(base, CH)], dbuf)

        def cg(g, _2):
            j = g // (SUB // 16)
            col = (g % (SUB // 16)) * 16
            sv = sbuf[pl.ds(g * 16, 16)]
            rv = rbuf[pl.ds(g * 16, 16)]
            dv = dbuf[pl.ds(g * 16, 16)]
            ov = jnp.where(c == 0, dv, sv)
            r8 = rv + jnp.where(c == 0, 0, NRL)
            kbuf[j, pl.ds(col, 16)] = r8 * N + ov
            return _2
        lax.fori_loop(0, GRP, cg, None)
        descs = [
            pltpu.async_copy(deg_sh.at[kbuf.at[j]],
                             nbuf.at[pl.ds(j * SUB, SUB)], sem)
            for j in range(NSUB)
        ]
        for d in descs:
            d.wait()
        pltpu.sync_copy(nbuf, norm_h.at[pl.ds(c * E + base, CH)])
        return _
    lax.fori_loop(0, epw // CH, norm_chunk, None)


def _layer_body(E, N, NRL, key_by_rel,
                table_h, src_h, rel_h, dst_h, norm_h, part_h,
                acc_sh, sbuf, rbuf, dbuf, kbuf, obuf, nbuf, rows, sem):
    c = lax.axis_index("c")
    s = lax.axis_index("s")
    iot = lax.iota(jnp.int32, 16)
    epw = E // NS
    rp = (N // NS + 7) // 8 * 8
    rlast = N - (NS - 1) * rp

    def zg(e, _):
        rows[e] = jnp.zeros((16,), jnp.float32)
        return _
    lax.fori_loop(0, CH, zg, None)

    @pl.when(s < NS - 1)
    def _zmain():
        pltpu.sync_copy(rows, acc_sh.at[pl.ds(s * rp, CH)])
        pltpu.sync_copy(rows.at[pl.ds(0, rp - CH)],
                        acc_sh.at[pl.ds(s * rp + CH, rp - CH)])

    @pl.when(s == NS - 1)
    def _zlast():
        pltpu.sync_copy(rows, acc_sh.at[pl.ds((NS - 1) * rp, CH)])
        pltpu.sync_copy(rows.at[pl.ds(0, rlast - CH)],
                        acc_sh.at[pl.ds((NS - 1) * rp + CH, rlast - CH)])

    plsc.subcore_barrier()

    def chunk(k, _):
        base = s * epw + k * CH
        pltpu.sync_copy(src_h.at[pl.ds(base, CH)], sbuf)
        pltpu.sync_copy(rel_h.at[pl.ds(base, CH)], rbuf)
        pltpu.sync_copy(dst_h.at[pl.ds(base, CH)], dbuf)
        pltpu.sync_copy(norm_h.at[pl.ds(c * E + base, CH)], nbuf)

        def cg(g, _2):
            j = g // (SUB // 16)
            col = (g % (SUB // 16)) * 16
            sv = sbuf[pl.ds(g * 16, 16)]
            rv = rbuf[pl.ds(g * 16, 16)]
            dv = dbuf[pl.ds(g * 16, 16)]
            gs = jnp.where(c == 0, sv, dv)
            ov = jnp.where(c == 0, dv, sv)
            r8 = rv + jnp.where(c == 0, 0, NRL)
            if key_by_rel:
                kg = r8 * N + gs
            else:
                kg = gs * 9 + r8
            kbuf[j, pl.ds(col, 16)] = kg
            obuf[j, pl.ds(col, 16)] = ov
            return _2
        lax.fori_loop(0, GRP, cg, None)

        descs = [
            pltpu.async_copy(table_h.at[kbuf.at[j]],
                             rows.at[pl.ds(j * SUB, SUB)], sem)
            for j in range(NSUB)
        ]
        for d in descs:
            d.wait()

        def scale(g, _2):
            nv = nbuf[pl.ds(g * 16, 16)]
            for i in range(16):
                e = g * 16 + i
                rows[e] = rows[e] * nv[i]
            return _2
        lax.fori_loop(0, GRP, scale, None)

        for j in range(NSUB):
            pltpu.sync_copy(rows.at[pl.ds(j * SUB, SUB)],
                            acc_sh.at[obuf.at[j]], add=True)
        return _
    lax.fori_loop(0, epw // CH, chunk, None)
    plsc.subcore_barrier()

    @pl.when(s < NS - 1)
    def _wmain():
        pltpu.sync_copy(acc_sh.at[pl.ds(s * rp, rp)],
                        part_h.at[c, pl.ds(s * rp, rp)])

    @pl.when(s == NS - 1)
    def _wlast():
        pltpu.sync_copy(acc_sh.at[pl.ds((NS - 1) * rp, rlast)],
                        part_h.at[c, pl.ds((NS - 1) * rp, rlast)])


def _mm_body(hp_ref, w18_ref, w2r_ref, xw_ref):
    h = jnp.maximum(hp_ref[0] + hp_ref[1] + w18_ref[...], 0.0)
    xw_ref[...] = jnp.dot(h, w2r_ref[...], preferred_element_type=jnp.float32)


def _fin_body(op_ref, xw_ref, out_ref):
    out_ref[...] = op_ref[0] + op_ref[1] + xw_ref[:, 8 * F:9 * F]


def kernel(W1, W2, src, rel, dst):
    R, N, _ = W1.shape
    NRL = (R - 1) // 2
    E = src.shape[0]
    DEGP = ((R - 1) * N + NS * 16 - 1) // (NS * 16) * (NS * 16)
    f32 = jnp.float32

    src = src.astype(jnp.int32)
    rel = rel.astype(jnp.int32)
    dst = dst.astype(jnp.int32)

    mesh = plsc.VectorSubcoreMesh(core_axis_name="c", subcore_axis_name="s")

    sc_params = pltpu.CompilerParams(use_tc_tiling_on_sc=False)

    norm_k = pl.kernel(
        functools.partial(_norm_body, E, N, NRL, DEGP),
        out_type=jax.ShapeDtypeStruct((2 * E,), f32),
        mesh=mesh,
        compiler_params=sc_params,
        scratch_types=[
            pltpu.VMEM_SHARED((DEGP,), f32),
            pltpu.VMEM((CH,), jnp.int32),
            pltpu.VMEM((CH,), jnp.int32),
            pltpu.VMEM((CH,), jnp.int32),
            pltpu.VMEM((NSUB, SUB), jnp.int32),
            pltpu.VMEM((CH,), f32),
            pltpu.VMEM((SUB,), f32),
            pltpu.VMEM((DEGP // NS,), f32),
            pltpu.SemaphoreType.DMA,
        ],
    )
    norm = norm_k(src, rel, dst)

    def layer(table, key_by_rel):
        lk = pl.kernel(
            functools.partial(_layer_body, E, N, NRL, key_by_rel),
            out_type=jax.ShapeDtypeStruct((NC, N, F), f32),
            mesh=mesh,
            compiler_params=sc_params,
            scratch_types=[
                pltpu.VMEM_SHARED((N, F), f32),
                pltpu.VMEM((CH,), jnp.int32),
                pltpu.VMEM((CH,), jnp.int32),
                pltpu.VMEM((CH,), jnp.int32),
                pltpu.VMEM((NSUB, SUB), jnp.int32),
                pltpu.VMEM((NSUB, SUB), jnp.int32),
                pltpu.VMEM((CH,), f32),
                pltpu.VMEM((CH, F), f32),
                pltpu.SemaphoreType.DMA,
            ],
        )
        return lk(table, src, rel, dst, norm)

    hparts = layer(W1.reshape(R * N, F), True)

    W2r = W2.transpose(1, 0, 2).reshape(F, R * F)
    BN = 2000
    xw = pl.pallas_call(
        _mm_body,
        grid=(N // BN,),
        in_specs=[
            pl.BlockSpec((NC, BN, F), lambda i: (0, i, 0)),
            pl.BlockSpec((BN, F), lambda i: (i, 0)),
            pl.BlockSpec((F, R * F), lambda i: (0, 0)),
        ],
        out_specs=pl.BlockSpec((BN, R * F), lambda i: (i, 0)),
        out_shape=jax.ShapeDtypeStruct((N, R * F), f32),
    )(hparts, W1[R - 1], W2r)

    oparts = layer(xw.reshape(N * R, F), False)

    out = pl.pallas_call(
        _fin_body,
        grid=(N // BN,),
        in_specs=[
            pl.BlockSpec((NC, BN, F), lambda i: (0, i, 0)),
            pl.BlockSpec((BN, R * F), lambda i: (i, 0)),
        ],
        out_specs=pl.BlockSpec((BN, F), lambda i: (i, 0)),
        out_shape=jax.ShapeDtypeStruct((N, F), f32),
    )(oparts, xw)
    return out


# speedup vs baseline: 116.6591x; 1.6511x over previous
import functools

import jax
import jax.numpy as jnp
from jax import lax
from jax.experimental import pallas as pl
from jax.experimental.pallas import tpu as pltpu
from jax.experimental.pallas import tpu_sc as plsc

F = 16
NS = 16
NC = 2
SUB = 80
NSUB = 25
CH = SUB * NSUB
SG = SUB // 16


def _zero_rows(rows):
    def zg(e, carry):
        rows[e] = jnp.zeros((16,), jnp.float32)
        return carry
    lax.fori_loop(0, CH, zg, None)


def _zero_acc(s, rows, acc_sh, n):
    rp = (n // NS + 7) // 8 * 8
    rlast = n - (NS - 1) * rp

    @pl.when(s < NS - 1)
    def _zmain():
        pltpu.sync_copy(rows, acc_sh.at[pl.ds(s * rp, CH)])
        pltpu.sync_copy(rows.at[pl.ds(0, rp - CH)],
                        acc_sh.at[pl.ds(s * rp + CH, rp - CH)])

    @pl.when(s == NS - 1)
    def _zlast():
        pltpu.sync_copy(rows, acc_sh.at[pl.ds((NS - 1) * rp, CH)])
        pltpu.sync_copy(rows.at[pl.ds(0, rlast - CH)],
                        acc_sh.at[pl.ds((NS - 1) * rp + CH, rlast - CH)])


def _write_acc(c, s, acc_sh, part_h, n):
    rp = (n // NS + 7) // 8 * 8
    rlast = n - (NS - 1) * rp

    @pl.when(s < NS - 1)
    def _wmain():
        pltpu.sync_copy(acc_sh.at[pl.ds(s * rp, rp)],
                        part_h.at[c, pl.ds(s * rp, rp)])

    @pl.when(s == NS - 1)
    def _wlast():
        pltpu.sync_copy(acc_sh.at[pl.ds((NS - 1) * rp, rlast)],
                        part_h.at[c, pl.ds((NS - 1) * rp, rlast)])


def _norm_body(E, N, NRL, DEGP,
               src_h, rel_h, dst_h, norm_h,
               deg_sh, sbuf, rbuf, dbuf, knbuf,
               nbuf, ones_v, work_v, seml, semg, sems):
    c = lax.axis_index("c")
    s = lax.axis_index("s")
    slc = DEGP // NS
    epw = E // NS
    eps = (2 * E) // NS

    def fz(g, carry):
        work_v[pl.ds(g * 16, 16)] = jnp.zeros((16,), jnp.float32)
        return carry
    lax.fori_loop(0, slc // 16, fz, None)
    for j in range(SUB // 16):
        ones_v[pl.ds(j * 16, 16)] = jnp.ones((16,), jnp.float32)
    pltpu.sync_copy(work_v, deg_sh.at[pl.ds(s * slc, slc)])
    plsc.subcore_barrier()

    fwd = s < 8
    base0 = jnp.where(fwd, s, s - 8) * eps

    def deg_chunk(k, carry):
        base = base0 + k * CH
        dl = [pltpu.async_copy(src_h.at[pl.ds(base, CH)], sbuf, seml),
              pltpu.async_copy(rel_h.at[pl.ds(base, CH)], rbuf, seml),
              pltpu.async_copy(dst_h.at[pl.ds(base, CH)], dbuf, seml)]
        for d in dl:
            d.wait()
        scat = []
        for j in range(NSUB):
            def cg(gi, carry2):
                off = j * SUB + gi * 16
                sv = sbuf[pl.ds(off, 16)]
                rv = rbuf[pl.ds(off, 16)]
                dv = dbuf[pl.ds(off, 16)]
                ov = jnp.where(fwd, dv, sv)
                r8 = rv + jnp.where(fwd, 0, NRL)
                knbuf[j, pl.ds(gi * 16, 16)] = r8 * N + ov
                return carry2
            lax.fori_loop(0, SG, cg, None)
            scat.append(pltpu.async_copy(ones_v, deg_sh.at[knbuf.at[j]],
                                         sems, add=True))
        for d in scat:
            d.wait()
        return carry
    lax.fori_loop(0, eps // CH, deg_chunk, None)
    plsc.subcore_barrier()

    off = s * slc
    pltpu.sync_copy(deg_sh.at[pl.ds(off, slc)], work_v)

    def inv_g(g, carry):
        v = work_v[pl.ds(g * 16, 16)]
        work_v[pl.ds(g * 16, 16)] = 1.0 / jnp.maximum(v, 1.0)
        return carry
    lax.fori_loop(0, slc // 16, inv_g, None)
    pltpu.sync_copy(work_v, deg_sh.at[pl.ds(off, slc)])
    plsc.subcore_barrier()

    def norm_chunk(k, carry):
        base = s * epw + k * CH
        dl = [pltpu.async_copy(src_h.at[pl.ds(base, CH)], sbuf, seml),
              pltpu.async_copy(rel_h.at[pl.ds(base, CH)], rbuf, seml),
              pltpu.async_copy(dst_h.at[pl.ds(base, CH)], dbuf, seml)]
        for d in dl:
            d.wait()
        gat = []
        for j in range(NSUB):
            def cg(gi, carry2):
                off = j * SUB + gi * 16
                sv = sbuf[pl.ds(off, 16)]
                rv = rbuf[pl.ds(off, 16)]
                dv = dbuf[pl.ds(off, 16)]
                ov = jnp.where(c == 0, dv, sv)
                r8 = rv + jnp.where(c == 0, 0, NRL)
                knbuf[j, pl.ds(gi * 16, 16)] = r8 * N + ov
                return carry2
            lax.fori_loop(0, SG, cg, None)
            gat.append(pltpu.async_copy(deg_sh.at[knbuf.at[j]],
                                        nbuf.at[pl.ds(j * SUB, SUB)],
                                        semg.at[j]))
        for d in gat:
            d.wait()
        pltpu.sync_copy(nbuf, norm_h.at[pl.ds(c * E + base, CH)])
        return carry
    lax.fori_loop(0, epw // CH, norm_chunk, None)


def _layer_body(E, N, NRL, key_by_rel,
                table_h, src_h, rel_h, dst_h, norm_h, part_h,
                acc_sh, sbuf, rbuf, dbuf, kbuf, obuf,
                nbuf, rows, seml, semg, sems):
    c = lax.axis_index("c")
    s = lax.axis_index("s")
    epw = E // NS

    _zero_rows(rows)
    _zero_acc(s, rows, acc_sh, N)
    plsc.subcore_barrier()

    def chunk(k, carry):
        base = s * epw + k * CH
        dl = [pltpu.async_copy(src_h.at[pl.ds(base, CH)], sbuf, seml),
              pltpu.async_copy(rel_h.at[pl.ds(base, CH)], rbuf, seml),
              pltpu.async_copy(dst_h.at[pl.ds(base, CH)], dbuf, seml),
              pltpu.async_copy(norm_h.at[pl.ds(c * E + base, CH)],
                               nbuf, seml)]
        for d in dl:
            d.wait()
        gat = []
        for j in range(NSUB):
            def cg(gi, carry2):
                off = j * SUB + gi * 16
                sv = sbuf[pl.ds(off, 16)]
                rv = rbuf[pl.ds(off, 16)]
                dv = dbuf[pl.ds(off, 16)]
                gs = jnp.where(c == 0, sv, dv)
                ov = jnp.where(c == 0, dv, sv)
                r8 = rv + jnp.where(c == 0, 0, NRL)
                if key_by_rel:
                    kg = r8 * N + gs
                else:
                    kg = gs * 9 + r8
                kbuf[j, pl.ds(gi * 16, 16)] = kg
                obuf[j, pl.ds(gi * 16, 16)] = ov
                return carry2
            lax.fori_loop(0, SG, cg, None)
            gat.append(pltpu.async_copy(table_h.at[kbuf.at[j]],
                                        rows.at[pl.ds(j * SUB, SUB)],
                                        semg.at[j]))
        scat = []
        for j in range(NSUB):
            gat[j].wait()

            def scale(gi, carry2):
                off = j * SUB + gi * 16
                nv = nbuf[pl.ds(off, 16)]
                for i in range(16):
                    rows[off + i] = rows[off + i] * nv[i]
                return carry2
            lax.fori_loop(0, SG, scale, None)
            scat.append(pltpu.async_copy(rows.at[pl.ds(j * SUB, SUB)],
                                         acc_sh.at[obuf.at[j]], sems,
                                         add=True))
        for d in scat:
            d.wait()
        return carry
    lax.fori_loop(0, epw // CH, chunk, None)
    plsc.subcore_barrier()
    _write_acc(c, s, acc_sh, part_h, N)


def _mm_body(hp_ref, w18_ref, w2r_ref, xw_ref):
    h = jnp.maximum(hp_ref[0] + hp_ref[1] + w18_ref[...], 0.0)
    xw_ref[...] = jnp.dot(h, w2r_ref[...], preferred_element_type=jnp.float32)


def _fin_body(op_ref, xw_ref, out_ref):
    out_ref[...] = op_ref[0] + op_ref[1] + xw_ref[:, 8 * F:9 * F]


def kernel(W1, W2, src, rel, dst):
    R, N, _ = W1.shape
    NRL = (R - 1) // 2
    E = src.shape[0]
    DEGP = ((R - 1) * N + NS * 16 - 1) // (NS * 16) * (NS * 16)
    f32 = jnp.float32

    src = src.astype(jnp.int32)
    rel = rel.astype(jnp.int32)
    dst = dst.astype(jnp.int32)

    mesh = plsc.VectorSubcoreMesh(core_axis_name="c", subcore_axis_name="s")
    sc_params = pltpu.CompilerParams(use_tc_tiling_on_sc=False)

    norm_k = pl.kernel(
        functools.partial(_norm_body, E, N, NRL, DEGP),
        out_type=jax.ShapeDtypeStruct((2 * E,), f32),
        mesh=mesh,
        compiler_params=sc_params,
        scratch_types=[
            pltpu.VMEM_SHARED((DEGP,), f32),
            pltpu.VMEM((CH,), jnp.int32),
            pltpu.VMEM((CH,), jnp.int32),
            pltpu.VMEM((CH,), jnp.int32),
            pltpu.VMEM((NSUB, SUB), jnp.int32),
            pltpu.VMEM((CH,), f32),
            pltpu.VMEM((SUB,), f32),
            pltpu.VMEM((DEGP // NS,), f32),
            pltpu.SemaphoreType.DMA,
            pltpu.SemaphoreType.DMA((NSUB,)),
            pltpu.SemaphoreType.DMA,
        ],
    )
    norm = norm_k(src, rel, dst)

    def layer(table, key_by_rel):
        lk = pl.kernel(
            functools.partial(_layer_body, E, N, NRL, key_by_rel),
            out_type=jax.ShapeDtypeStruct((NC, N, F), f32),
            mesh=mesh,
            compiler_params=sc_params,
            scratch_types=[
                pltpu.VMEM_SHARED((N, F), f32),
                pltpu.VMEM((CH,), jnp.int32),
                pltpu.VMEM((CH,), jnp.int32),
                pltpu.VMEM((CH,), jnp.int32),
                pltpu.VMEM((NSUB, SUB), jnp.int32),
                pltpu.VMEM((NSUB, SUB), jnp.int32),
                pltpu.VMEM((CH,), f32),
                pltpu.VMEM((CH, F), f32),
                pltpu.SemaphoreType.DMA,
                pltpu.SemaphoreType.DMA((NSUB,)),
                pltpu.SemaphoreType.DMA,
            ],
        )
        return lk(table, src, rel, dst, norm)

    hparts = layer(W1.reshape(R * N, F), True)

    W2r = W2.transpose(1, 0, 2).reshape(F, R * F)
    BN = 2000
    xw = pl.pallas_call(
        _mm_body,
        grid=(N // BN,),
        in_specs=[
            pl.BlockSpec((NC, BN, F), lambda i: (0, i, 0)),
            pl.BlockSpec((BN, F), lambda i: (i, 0)),
            pl.BlockSpec((F, R * F), lambda i: (0, 0)),
        ],
        out_specs=pl.BlockSpec((BN, R * F), lambda i: (i, 0)),
        out_shape=jax.ShapeDtypeStruct((N, R * F), f32),
    )(hparts, W1[R - 1], W2r)

    oparts = layer(xw.reshape(N * R, F), False)

    out = pl.pallas_call(
        _fin_body,
        grid=(N // BN,),
        in_specs=[
            pl.BlockSpec((NC, BN, F), lambda i: (0, i, 0)),
            pl.BlockSpec((BN, R * F), lambda i: (i, 0)),
        ],
        out_specs=pl.BlockSpec((BN, F), lambda i: (i, 0)),
        out_shape=jax.ShapeDtypeStruct((N, F), f32),
    )(oparts, xw)
    return out


# speedup vs baseline: 122.2274x; 1.0477x over previous
import functools

import jax
import jax.numpy as jnp
from jax import lax
from jax.experimental import pallas as pl
from jax.experimental.pallas import tpu as pltpu
from jax.experimental.pallas import tpu_sc as plsc

F = 16
NS = 16
NC = 2
SUB = 80
NSUB = 25
CH = SUB * NSUB
SG = SUB // 16


def _pieces(total):
    return [CH] * (total // CH) + ([total % CH] if total % CH else [])


def _zero_rows(rows):
    def zg(e, carry):
        rows[e] = jnp.zeros((16,), jnp.float32)
        return carry
    lax.fori_loop(0, CH, zg, None)


def _zero_acc(s, rows, acc_sh, n):
    rp = (n // NS + 7) // 8 * 8
    rlast = n - (NS - 1) * rp

    @pl.when(s < NS - 1)
    def _zmain():
        pltpu.sync_copy(rows, acc_sh.at[pl.ds(s * rp, CH)])
        pltpu.sync_copy(rows.at[pl.ds(0, rp - CH)],
                        acc_sh.at[pl.ds(s * rp + CH, rp - CH)])

    @pl.when(s == NS - 1)
    def _zlast():
        pltpu.sync_copy(rows, acc_sh.at[pl.ds((NS - 1) * rp, CH)])
        pltpu.sync_copy(rows.at[pl.ds(0, rlast - CH)],
                        acc_sh.at[pl.ds((NS - 1) * rp + CH, rlast - CH)])


def _write_acc(c, s, acc_sh, part_h, n):
    rp = (n // NS + 7) // 8 * 8
    rlast = n - (NS - 1) * rp

    @pl.when(s < NS - 1)
    def _wmain():
        pltpu.sync_copy(acc_sh.at[pl.ds(s * rp, rp)],
                        part_h.at[c, pl.ds(s * rp, rp)])

    @pl.when(s == NS - 1)
    def _wlast():
        pltpu.sync_copy(acc_sh.at[pl.ds((NS - 1) * rp, rlast)],
                        part_h.at[c, pl.ds((NS - 1) * rp, rlast)])


def _load_ends(base, src_h, dst_h, gbuf, dbuf, seml):
    return [pltpu.async_copy(src_h.at[pl.ds(base, CH)], gbuf, seml),
            pltpu.async_copy(dst_h.at[pl.ds(base, CH)], dbuf, seml)]


def _norm_body(E, N, NRL, DEGP,
               src_h, rel_h, dst_h, norm_h,
               deg_sh, gbuf, rbuf, dbuf, knbuf,
               nbuf, ones_v, seml, semg, sems):
    c = lax.axis_index("c")
    s = lax.axis_index("s")
    slc = DEGP // NS
    epw = E // NS

    def fz(g, carry):
        nbuf[pl.ds(g * 16, 16)] = jnp.zeros((16,), jnp.float32)
        return carry
    lax.fori_loop(0, CH // 16, fz, None)
    for j in range(SUB // 16):
        ones_v[pl.ds(j * 16, 16)] = jnp.ones((16,), jnp.float32)
    poff = 0
    for psz in _pieces(slc):
        pltpu.sync_copy(nbuf.at[pl.ds(0, psz)],
                        deg_sh.at[pl.ds(s * slc + poff, psz)])
        poff += psz
    plsc.subcore_barrier()

    def deg_chunk(k, carry):
        base = s * epw + k * CH
        dl = [pltpu.async_copy(rel_h.at[pl.ds(base, CH)], rbuf, seml)]
        dl += _load_ends(base, src_h, dst_h, gbuf, dbuf, seml)
        for d in dl:
            d.wait()
        scat = []
        for j in range(NSUB):
            def cg(gi, carry2):
                off = j * SUB + gi * 16
                rv = rbuf[pl.ds(off, 16)]
                sv = gbuf[pl.ds(off, 16)]
                dv = dbuf[pl.ds(off, 16)]
                ov = jnp.where(c == 0, dv, sv)
                knbuf[j, pl.ds(gi * 16, 16)] = rv * N + ov
                return carry2
            lax.fori_loop(0, SG, cg, None)
            scat.append(pltpu.async_copy(ones_v, deg_sh.at[knbuf.at[j]],
                                         sems, add=True))
        for d in scat:
            d.wait()
        return carry
    lax.fori_loop(0, epw // CH, deg_chunk, None)
    plsc.subcore_barrier()

    off = s * slc
    poff = 0
    for psz in _pieces(slc):
        pltpu.sync_copy(deg_sh.at[pl.ds(off + poff, psz)],
                        nbuf.at[pl.ds(0, psz)])

        def inv_g(g, carry):
            v = nbuf[pl.ds(g * 16, 16)]
            nbuf[pl.ds(g * 16, 16)] = 1.0 / jnp.maximum(v, 1.0)
            return carry
        lax.fori_loop(0, psz // 16, inv_g, None)
        pltpu.sync_copy(nbuf.at[pl.ds(0, psz)],
                        deg_sh.at[pl.ds(off + poff, psz)])
        poff += psz
    plsc.subcore_barrier()

    def norm_chunk(k, carry):
        base = s * epw + k * CH
        dl = [pltpu.async_copy(rel_h.at[pl.ds(base, CH)], rbuf, seml)]
        dl += _load_ends(base, src_h, dst_h, gbuf, dbuf, seml)
        for d in dl:
            d.wait()
        gat = []
        for j in range(NSUB):
            def cg(gi, carry2):
                off = j * SUB + gi * 16
                rv = rbuf[pl.ds(off, 16)]
                sv = gbuf[pl.ds(off, 16)]
                dv = dbuf[pl.ds(off, 16)]
                ov = jnp.where(c == 0, dv, sv)
                knbuf[j, pl.ds(gi * 16, 16)] = rv * N + ov
                return carry2
            lax.fori_loop(0, SG, cg, None)
            gat.append(pltpu.async_copy(deg_sh.at[knbuf.at[j]],
                                        nbuf.at[pl.ds(j * SUB, SUB)],
                                        semg.at[j]))
        for d in gat:
            d.wait()
        pltpu.sync_copy(nbuf, norm_h.at[pl.ds(c * E + base, CH)])
        return carry
    lax.fori_loop(0, epw // CH, norm_chunk, None)


def _layer_body(E, N, NRL, key_by_rel,
                table_h, src_h, rel_h, dst_h, norm_h, part_h,
                acc_sh, gbuf, rbuf, dbuf, kbuf, obuf,
                nbuf, rows, seml, semg, sems):
    c = lax.axis_index("c")
    s = lax.axis_index("s")
    epw = E // NS

    _zero_rows(rows)
    _zero_acc(s, rows, acc_sh, N)
    plsc.subcore_barrier()

    if key_by_rel:
        koff = c * (NRL * N)
    else:
        koff = c * NRL

    def chunk(k, carry):
        base = s * epw + k * CH
        dl = [pltpu.async_copy(rel_h.at[pl.ds(base, CH)], rbuf, seml),
              pltpu.async_copy(norm_h.at[pl.ds(c * E + base, CH)],
                               nbuf, seml)]
        dl += _load_ends(base, src_h, dst_h, gbuf, dbuf, seml)
        for d in dl:
            d.wait()
        gat = []
        for j in range(NSUB):
            def cg(gi, carry2):
                off = j * SUB + gi * 16
                sv = gbuf[pl.ds(off, 16)]
                rv = rbuf[pl.ds(off, 16)]
                dv = dbuf[pl.ds(off, 16)]
                gs = jnp.where(c == 0, sv, dv)
                ov = jnp.where(c == 0, dv, sv)
                if key_by_rel:
                    kg = rv * N + gs + koff
                else:
                    kg = gs * 9 + rv + koff
                kbuf[j, pl.ds(gi * 16, 16)] = kg
                obuf[j, pl.ds(gi * 16, 16)] = ov
                return carry2
            lax.fori_loop(0, SG, cg, None)
            gat.append(pltpu.async_copy(table_h.at[kbuf.at[j]],
                                        rows.at[pl.ds(j * SUB, SUB)],
                                        semg.at[j]))
        scat = []
        for j in range(NSUB):
            gat[j].wait()

            def scale(gi, carry2):
                off = j * SUB + gi * 16
                nv = nbuf[pl.ds(off, 16)]
                for i in range(16):
                    rows[off + i] = rows[off + i] * nv[i]
                return carry2
            lax.fori_loop(0, SG, scale, None)
            scat.append(pltpu.async_copy(rows.at[pl.ds(j * SUB, SUB)],
                                         acc_sh.at[obuf.at[j]], sems,
                                         add=True))
        for d in scat:
            d.wait()
        return carry
    lax.fori_loop(0, epw // CH, chunk, None)
    plsc.subcore_barrier()
    _write_acc(c, s, acc_sh, part_h, N)


def _mm_body(hp_ref, w18_ref, w2r_ref, xw_ref):
    h = jnp.maximum(hp_ref[0] + hp_ref[1] + w18_ref[...], 0.0)
    xw_ref[...] = jnp.dot(h, w2r_ref[...], preferred_element_type=jnp.float32)


def _fin_body(op_ref, xw_ref, out_ref):
    out_ref[...] = op_ref[0] + op_ref[1] + xw_ref[:, 8 * F:9 * F]


def kernel(W1, W2, src, rel, dst):
    R, N, _ = W1.shape
    NRL = (R - 1) // 2
    E = src.shape[0]
    DEGP = (NRL * N + NS * 16 - 1) // (NS * 16) * (NS * 16)
    f32 = jnp.float32

    src = src.astype(jnp.int32)
    rel = rel.astype(jnp.int32)
    dst = dst.astype(jnp.int32)

    mesh = plsc.VectorSubcoreMesh(core_axis_name="c", subcore_axis_name="s")
    sc_params = pltpu.CompilerParams(use_tc_tiling_on_sc=False)

    norm_k = pl.kernel(
        functools.partial(_norm_body, E, N, NRL, DEGP),
        out_type=jax.ShapeDtypeStruct((2 * E,), f32),
        mesh=mesh,
        compiler_params=sc_params,
        scratch_types=[
            pltpu.VMEM_SHARED((DEGP,), f32),
            pltpu.VMEM((CH,), jnp.int32),
            pltpu.VMEM((CH,), jnp.int32),
            pltpu.VMEM((CH,), jnp.int32),
            pltpu.VMEM((NSUB, SUB), jnp.int32),
            pltpu.VMEM((CH,), f32),
            pltpu.VMEM((SUB,), f32),
            pltpu.SemaphoreType.DMA,
            pltpu.SemaphoreType.DMA((NSUB,)),
            pltpu.SemaphoreType.DMA,
        ],
    )
    norm = norm_k(src, rel, dst)

    def layer(table, key_by_rel):
        lk = pl.kernel(
            functools.partial(_layer_body, E, N, NRL, key_by_rel),
            out_type=jax.ShapeDtypeStruct((NC, N, F), f32),
            mesh=mesh,
            compiler_params=sc_params,
            scratch_types=[
                pltpu.VMEM_SHARED((N, F), f32),
                pltpu.VMEM((CH,), jnp.int32),
                pltpu.VMEM((CH,), jnp.int32),
                pltpu.VMEM((CH,), jnp.int32),
                pltpu.VMEM((NSUB, SUB), jnp.int32),
                pltpu.VMEM((NSUB, SUB), jnp.int32),
                pltpu.VMEM((CH,), f32),
                pltpu.VMEM((CH, F), f32),
                pltpu.SemaphoreType.DMA,
                pltpu.SemaphoreType.DMA((NSUB,)),
                pltpu.SemaphoreType.DMA,
            ],
        )
        return lk(table, src, rel, dst, norm)

    hparts = layer(W1.reshape(R * N, F), True)

    W2r = W2.transpose(1, 0, 2).reshape(F, R * F)
    BN = 2000
    xw = pl.pallas_call(
        _mm_body,
        grid=(N // BN,),
        in_specs=[
            pl.BlockSpec((NC, BN, F), lambda i: (0, i, 0)),
            pl.BlockSpec((BN, F), lambda i: (i, 0)),
            pl.BlockSpec((F, R * F), lambda i: (0, 0)),
        ],
        out_specs=pl.BlockSpec((BN, R * F), lambda i: (i, 0)),
        out_shape=jax.ShapeDtypeStruct((N, R * F), f32),
    )(hparts, W1[R - 1], W2r)

    oparts = layer(xw.reshape(N * R, F), False)

    out = pl.pallas_call(
        _fin_body,
        grid=(N // BN,),
        in_specs=[
            pl.BlockSpec((NC, BN, F), lambda i: (0, i, 0)),
            pl.BlockSpec((BN, R * F), lambda i: (i, 0)),
        ],
        out_specs=pl.BlockSpec((BN, F), lambda i: (i, 0)),
        out_shape=jax.ShapeDtypeStruct((N, F), f32),
    )(oparts, xw)
    return out
